# baseline (speedup 1.0000x reference)
import functools

import jax
import jax.numpy as jnp
from jax import lax
from jax.experimental import pallas as pl
from jax.experimental.pallas import tpu as pltpu
from jax.experimental.pallas import tpu_sc as plsc

_NC, _NS = 2, 16
_NW = _NC * _NS
_CHUNK = 128
_BLK = 2048


def _make_sc_gather(n_pad: int, emb: int, n_chunks: int):
    b_per_w = n_chunks * _CHUNK
    mesh = plsc.VectorSubcoreMesh(
        core_axis_name="c", subcore_axis_name="s",
        num_cores=_NC, num_subcores=_NS,
    )

    @functools.partial(
        pl.kernel,
        mesh=mesh,
        compiler_params=pltpu.CompilerParams(use_tc_tiling_on_sc=False),
        out_type=jax.ShapeDtypeStruct((n_pad, emb), jnp.float32),
        scratch_types=[
            pltpu.VMEM((b_per_w,), jnp.int32),
            pltpu.VMEM((b_per_w, emb), jnp.float32),
            pltpu.SemaphoreType.DMA,
        ],
    )
    def gather(idx_hbm, table_hbm, out_hbm, idx_v, rows_v, sem):
        wid = lax.axis_index("s") * _NC + lax.axis_index("c")
        pltpu.sync_copy(idx_hbm.at[pl.ds(wid * b_per_w, b_per_w)], idx_v)

        def body(j, carry):
            pltpu.async_copy(
                table_hbm.at[idx_v.at[pl.ds(j * _CHUNK, _CHUNK)]],
                rows_v.at[pl.ds(j * _CHUNK, _CHUNK)],
                sem,
            )
            return carry

        lax.fori_loop(0, n_chunks, body, 0)
        pltpu.make_async_copy(
            out_hbm.at[pl.ds(wid * b_per_w, b_per_w)], rows_v, sem
        ).wait()
        pltpu.sync_copy(rows_v, out_hbm.at[pl.ds(wid * b_per_w, b_per_w)])

    return gather


def _tc_body(e_ref, f_ref, d_ref, c_ref, we_ref, wf_ref, wd_ref, wc_ref,
             b_ref, g_ref, bet_ref, out_ref):
    h = jnp.dot(e_ref[...], we_ref[...], preferred_element_type=jnp.float32)
    h = h + jnp.dot(f_ref[...], wf_ref[...], preferred_element_type=jnp.float32)
    h = h + jnp.dot(d_ref[...], wd_ref[...], preferred_element_type=jnp.float32)
    h = h + jnp.dot(c_ref[...], wc_ref[...], preferred_element_type=jnp.float32)
    h = h + b_ref[...]
    h = h * (1.0 / (1.0 + jnp.exp(-h)))
    mu = jnp.mean(h, axis=-1, keepdims=True)
    hc = h - mu
    var = jnp.mean(hc * hc, axis=-1, keepdims=True)
    out_ref[...] = hc * lax.rsqrt(var + 1e-5) * g_ref[...] + bet_ref[...]


def _sc_gather(oc, table, n_pad, n_chunks):
    return _make_sc_gather(n_pad, table.shape[1], n_chunks)(oc, table)


def kernel(op_code, features, configs, dim_features, table, W, b, gamma, beta):
    n = features.shape[0]
    ne = table.shape[1]
    nf = features.shape[1]
    nd = dim_features.shape[1]
    ncf = configs.shape[1]
    out_ch = W.shape[1]

    n_chunks = -(-n // (_NW * _CHUNK))
    n_pad = _NW * _CHUNK * n_chunks
    oc = op_code.reshape(-1).astype(jnp.int32)
    oc = jnp.concatenate([oc, jnp.zeros((n_pad - n,), jnp.int32)])

    e = _sc_gather(oc, table, n_pad, n_chunks)

    w_e = W[:ne]
    w_f = W[ne:ne + nf]
    w_d = W[ne + nf:ne + nf + nd]
    w_c = W[ne + nf + nd:]

    grid = (-(-n // _BLK),)
    row_block = lambda width: pl.BlockSpec((_BLK, width), lambda i: (i, 0))
    full = lambda a: pl.BlockSpec(a.shape, lambda i: (0, 0))

    out = pl.pallas_call(
        _tc_body,
        grid=grid,
        in_specs=[
            row_block(ne),
            row_block(nf),
            row_block(nd),
            row_block(ncf),
            full(w_e), full(w_f), full(w_d), full(w_c),
            pl.BlockSpec((1, out_ch), lambda i: (0, 0)),
            pl.BlockSpec((1, out_ch), lambda i: (0, 0)),
            pl.BlockSpec((1, out_ch), lambda i: (0, 0)),
        ],
        out_specs=pl.BlockSpec((_BLK, out_ch), lambda i: (i, 0)),
        out_shape=jax.ShapeDtypeStruct((n, out_ch), jnp.float32),
    )(e, features, dim_features, configs, w_e, w_f, w_d, w_c,
      b.reshape(1, -1), gamma.reshape(1, -1), beta.reshape(1, -1))
    return out


# speedup vs baseline: 2.1075x; 2.1075x over previous
import functools

import jax
import jax.numpy as jnp
from jax import lax
from jax.experimental import pallas as pl
from jax.experimental.pallas import tpu as pltpu
from jax.experimental.pallas import tpu_sc as plsc

_NC, _NS = 2, 16
_NW = _NC * _NS
_CHUNK = 128
_BLK = 2048


def _make_sc_gather(n_pad: int, emb: int, n_chunks: int):
    b_per_w = n_chunks * _CHUNK
    mesh = plsc.VectorSubcoreMesh(
        core_axis_name="c", subcore_axis_name="s",
        num_cores=_NC, num_subcores=_NS,
    )

    pack_rows = b_per_w * emb // 128

    @functools.partial(
        pl.kernel,
        mesh=mesh,
        compiler_params=pltpu.CompilerParams(use_tc_tiling_on_sc=False),
        out_type=jax.ShapeDtypeStruct((n_pad, emb), jnp.float32),
        scratch_types=[
            pltpu.VMEM((b_per_w,), jnp.int32),
            pltpu.VMEM((b_per_w, emb), jnp.float32),
            pltpu.SemaphoreType.DMA,
        ],
    )
    def gather(idx_hbm, table_hbm, out_hbm, idx_v, rows_v, sem):
        wid = lax.axis_index("s") * _NC + lax.axis_index("c")
        pltpu.sync_copy(idx_hbm.at[pl.ds(wid * b_per_w, b_per_w)], idx_v)

        def body(j, carry):
            pltpu.async_copy(
                table_hbm.at[idx_v.at[pl.ds(j * _CHUNK, _CHUNK)]],
                rows_v.at[pl.ds(j * _CHUNK, _CHUNK)],
                sem,
            )
            return carry

        lax.fori_loop(0, n_chunks, body, 0)
        out_slice = out_hbm.at[pl.ds(wid * b_per_w, b_per_w)]
        pltpu.make_async_copy(out_slice, rows_v, sem).wait()
        pltpu.sync_copy(rows_v, out_slice)

    return gather


_DN0 = (((0,), (0,)), ((), ()))
_DN1 = (((0,), (1,)), ((), ()))


def _tc_body(e_ref, f_ref, d_ref, c_ref, we_ref, wf_ref, wd_ref, wc_ref,
             b_ref, g_ref, bet_ref, out_ref):
    h = lax.dot_general(wf_ref[...], f_ref[...], _DN0,
                        preferred_element_type=jnp.float32)
    h = h + lax.dot_general(wd_ref[...], d_ref[...], _DN0,
                            preferred_element_type=jnp.float32)
    h = h + lax.dot_general(wc_ref[...], c_ref[...], _DN0,
                            preferred_element_type=jnp.float32)
    e4 = e_ref[...]
    ne = we_ref.shape[0]
    h_e = jnp.concatenate(
        [lax.dot_general(we_ref[...],
                         lax.slice(e4, (0, a * ne), (e4.shape[0], (a + 1) * ne)),
                         _DN1, preferred_element_type=jnp.float32)
         for a in range(4)],
        axis=1)
    h = h + h_e
    h = h + b_ref[...]
    h = h * (1.0 / (1.0 + jnp.exp(-h)))
    mu = jnp.mean(h, axis=0, keepdims=True)
    hc = h - mu
    var = jnp.mean(hc * hc, axis=0, keepdims=True)
    out_ref[...] = hc * lax.rsqrt(var + 1e-5) * g_ref[...] + bet_ref[...]


def _sc_gather(oc, table, n_pad, n_chunks):
    return _make_sc_gather(n_pad, table.shape[1], n_chunks)(oc, table)


def kernel(op_code, features, configs, dim_features, table, W, b, gamma, beta):
    n = features.shape[0]
    ne = table.shape[1]
    nf = features.shape[1]
    nd = dim_features.shape[1]
    ncf = configs.shape[1]
    out_ch = W.shape[1]

    n_chunks = -(-n // (_NW * _CHUNK))
    n_pad = _NW * _CHUNK * n_chunks
    oc = op_code.reshape(-1).astype(jnp.int32)
    oc = jnp.concatenate([oc, jnp.zeros((n_pad - n,), jnp.int32)])

    s = jnp.arange(n_pad, dtype=jnp.int32)
    i = s // _BLK
    t = s % _BLK
    r = i * _BLK + (t % 4) * (_BLK // 4) + t // 4
    e = _sc_gather(oc[r], table, n_pad, n_chunks)
    e = e.reshape(n_pad * ne // 128, 128)

    w_e = W[:ne]
    w_f = W[ne:ne + nf]
    w_d = W[ne + nf:ne + nf + nd]
    w_c = W[ne + nf + nd:]

    f_t = features.T
    d_t = dim_features.T
    c_t = configs.T

    grid = (-(-n // _BLK),)
    col_block = lambda ch: pl.BlockSpec((ch, _BLK), lambda i: (0, i))
    full = lambda a: pl.BlockSpec(a.shape, lambda i: (0, 0))

    out_t = pl.pallas_call(
        _tc_body,
        grid=grid,
        in_specs=[
            pl.BlockSpec((_BLK * ne // 128, 128), lambda i: (i, 0)),
            col_block(nf),
            col_block(nd),
            col_block(ncf),
            full(w_e), full(w_f), full(w_d), full(w_c),
            pl.BlockSpec((out_ch, 1), lambda i: (0, 0)),
            pl.BlockSpec((out_ch, 1), lambda i: (0, 0)),
            pl.BlockSpec((out_ch, 1), lambda i: (0, 0)),
        ],
        out_specs=pl.BlockSpec((out_ch, _BLK), lambda i: (0, i)),
        out_shape=jax.ShapeDtypeStruct((out_ch, n), jnp.float32),
    )(e, f_t, d_t, c_t, w_e, w_f, w_d, w_c,
      b.reshape(-1, 1), gamma.reshape(-1, 1), beta.reshape(-1, 1))
    return out_t.T


# speedup vs baseline: 2.2509x; 1.0680x over previous
import functools

import jax
import jax.numpy as jnp
from jax import lax
from jax.experimental import pallas as pl
from jax.experimental.pallas import tpu as pltpu
from jax.experimental.pallas import tpu_sc as plsc

_NC, _NS = 2, 16
_NW = _NC * _NS
_CHUNK = 128
_BLK = 2048


def _make_sc_gather(n_pad: int, emb: int, n_chunks: int):
    b_per_w = n_chunks * _CHUNK
    mesh = plsc.VectorSubcoreMesh(
        core_axis_name="c", subcore_axis_name="s",
        num_cores=_NC, num_subcores=_NS,
    )

    pack_rows = b_per_w * emb // 128

    @functools.partial(
        pl.kernel,
        mesh=mesh,
        compiler_params=pltpu.CompilerParams(use_tc_tiling_on_sc=False),
        out_type=jax.ShapeDtypeStruct((n_pad, emb), jnp.float32),
        scratch_types=[
            pltpu.VMEM((b_per_w,), jnp.int32),
            pltpu.VMEM((b_per_w, emb), jnp.float32),
            pltpu.SemaphoreType.DMA,
        ],
    )
    def gather(idx_hbm, table_hbm, out_hbm, idx_v, rows_v, sem):
        wid = lax.axis_index("s") * _NC + lax.axis_index("c")
        pltpu.sync_copy(idx_hbm.at[pl.ds(wid * b_per_w, b_per_w)], idx_v)

        def body(j, carry):
            pltpu.async_copy(
                table_hbm.at[idx_v.at[pl.ds(j * _CHUNK, _CHUNK)]],
                rows_v.at[pl.ds(j * _CHUNK, _CHUNK)],
                sem,
            )
            return carry

        lax.fori_loop(0, n_chunks, body, 0)
        out_slice = out_hbm.at[pl.ds(wid * b_per_w, b_per_w)]
        pltpu.make_async_copy(out_slice, rows_v, sem).wait()
        pltpu.sync_copy(rows_v, out_slice)

    return gather


_DN0 = (((0,), (0,)), ((), ()))
_DN1 = (((0,), (1,)), ((), ()))


def _tc_body(e_ref, f_ref, d_ref, c_ref, we_ref, wf_ref, wd_ref, wc_ref,
             b_ref, g_ref, bet_ref, out_ref):
    h = lax.dot_general(wf_ref[...], f_ref[...], _DN0,
                        preferred_element_type=jnp.float32)
    h = h + lax.dot_general(wd_ref[...], d_ref[...], _DN0,
                            preferred_element_type=jnp.float32)
    h = h + lax.dot_general(wc_ref[...], c_ref[...], _DN0,
                            preferred_element_type=jnp.float32)
    e4 = e_ref[...]
    ne = we_ref.shape[0]
    h_e = jnp.concatenate(
        [lax.dot_general(we_ref[...],
                         lax.slice(e4, (0, a * ne), (e4.shape[0], (a + 1) * ne)),
                         _DN1, preferred_element_type=jnp.float32)
         for a in range(4)],
        axis=1)
    h = h + h_e
    h = h + b_ref[...]
    h = h * (1.0 / (1.0 + jnp.exp(-h)))
    mu = jnp.mean(h, axis=0, keepdims=True)
    hc = h - mu
    var = jnp.mean(hc * hc, axis=0, keepdims=True)
    out_ref[...] = hc * lax.rsqrt(var + 1e-5) * g_ref[...] + bet_ref[...]


def _sc_gather(oc, table, n_pad, n_chunks):
    return _make_sc_gather(n_pad, table.shape[1], n_chunks)(oc, table)


def _tc_body_carry(e_ref, f_ref, d_ref, c_ref, we_ref, wf_ref, wd_ref,
                   wc_ref, b_ref, g_ref, bet_ref, carry_ref, out_ref):
    del carry_ref
    _tc_body(e_ref, f_ref, d_ref, c_ref, we_ref, wf_ref, wd_ref, wc_ref,
             b_ref, g_ref, bet_ref, out_ref)


def kernel(op_code, features, configs, dim_features, table, W, b, gamma, beta):
    n = features.shape[0]
    ne = table.shape[1]
    nf = features.shape[1]
    nd = dim_features.shape[1]
    ncf = configs.shape[1]
    out_ch = W.shape[1]

    n_chunks = -(-n // (_NW * _CHUNK))
    n_pad = _NW * _CHUNK * n_chunks
    oc = op_code.reshape(-1).astype(jnp.int32)
    oc = jnp.concatenate([oc, jnp.zeros((n_pad - n,), jnp.int32)])

    s = jnp.arange(n_pad, dtype=jnp.int32)
    i = s // _BLK
    t = s % _BLK
    r = i * _BLK + (t % 4) * (_BLK // 4) + t // 4
    oc_perm = oc[r]

    w_e = W[:ne]
    w_f = W[ne:ne + nf]
    w_d = W[ne + nf:ne + nf + nd]
    w_c = W[ne + nf + nd:]

    f_t = features.T
    d_t = dim_features.T
    c_t = configs.T
    scalars = (b.reshape(-1, 1), gamma.reshape(-1, 1), beta.reshape(-1, 1))

    n_blocks = -(-n // _BLK)
    g_blocks = n_pad // _BLK
    bounds = [0, 12, 24, 36, g_blocks]

    out_t = None
    for si in range(len(bounds) - 1):
        b0, b1 = bounds[si], bounds[si + 1]
        n_sl = (b1 - b0) * _BLK
        e_s = _sc_gather(oc_perm[b0 * _BLK:b1 * _BLK], table, n_sl,
                         n_sl // (_NW * _CHUNK))
        e_s = e_s.reshape(n_sl * ne // 128, 128)

        tb1 = min(b1, n_blocks)
        col_block = lambda ch, b0=b0: pl.BlockSpec(
            (ch, _BLK), lambda i, b0=b0: (0, b0 + i))
        full = lambda a: pl.BlockSpec(a.shape, lambda i: (0, 0))
        in_specs = [
            pl.BlockSpec((_BLK * ne // 128, 128), lambda i: (i, 0)),
            col_block(nf),
            col_block(nd),
            col_block(ncf),
            full(w_e), full(w_f), full(w_d), full(w_c),
            pl.BlockSpec((out_ch, 1), lambda i: (0, 0)),
            pl.BlockSpec((out_ch, 1), lambda i: (0, 0)),
            pl.BlockSpec((out_ch, 1), lambda i: (0, 0)),
        ]
        args = (e_s, f_t, d_t, c_t, w_e, w_f, w_d, w_c) + scalars
        body = _tc_body
        alias = {}
        if out_t is not None:
            in_specs = in_specs + [pl.BlockSpec(memory_space=pl.ANY)]
            args = args + (out_t,)
            body = _tc_body_carry
            alias = {11: 0}
        out_t = pl.pallas_call(
            body,
            grid=(tb1 - b0,),
            in_specs=in_specs,
            out_specs=pl.BlockSpec((out_ch, _BLK),
                                   lambda i, b0=b0: (0, b0 + i)),
            out_shape=jax.ShapeDtypeStruct((out_ch, n), jnp.float32),
            input_output_aliases=alias,
        )(*args)
    return out_t.T


# speedup vs baseline: 2.3605x; 1.0487x over previous
import functools

import jax
import jax.numpy as jnp
from jax import lax
from jax.experimental import pallas as pl
from jax.experimental.pallas import tpu as pltpu
from jax.experimental.pallas import tpu_sc as plsc

_NC, _NS = 2, 16
_NW = _NC * _NS
_CHUNK = 128
_BLK = 4096


def _make_sc_gather(n_pad: int, emb: int, n_chunks: int):
    b_per_w = n_chunks * _CHUNK
    mesh = plsc.VectorSubcoreMesh(
        core_axis_name="c", subcore_axis_name="s",
        num_cores=_NC, num_subcores=_NS,
    )

    pack_rows = b_per_w * emb // 128

    @functools.partial(
        pl.kernel,
        mesh=mesh,
        compiler_params=pltpu.CompilerParams(use_tc_tiling_on_sc=False),
        out_type=jax.ShapeDtypeStruct((n_pad, emb), jnp.float32),
        scratch_types=[
            pltpu.VMEM((b_per_w,), jnp.int32),
            pltpu.VMEM((b_per_w, emb), jnp.float32),
            pltpu.SemaphoreType.DMA,
        ],
    )
    def gather(idx_hbm, table_hbm, out_hbm, idx_v, rows_v, sem):
        wid = lax.axis_index("s") * _NC + lax.axis_index("c")
        pltpu.sync_copy(idx_hbm.at[pl.ds(wid * b_per_w, b_per_w)], idx_v)

        def body(j, carry):
            pltpu.async_copy(
                table_hbm.at[idx_v.at[pl.ds(j * _CHUNK, _CHUNK)]],
                rows_v.at[pl.ds(j * _CHUNK, _CHUNK)],
                sem,
            )
            return carry

        lax.fori_loop(0, n_chunks, body, 0)
        out_slice = out_hbm.at[pl.ds(wid * b_per_w, b_per_w)]
        pltpu.make_async_copy(out_slice, rows_v, sem).wait()
        pltpu.sync_copy(rows_v, out_slice)

    return gather


_DN0 = (((0,), (0,)), ((), ()))
_DN1 = (((0,), (1,)), ((), ()))


def _tc_body(e_ref, f_ref, d_ref, c_ref, we_ref, wf_ref, wd_ref, wc_ref,
             b_ref, g_ref, bet_ref, out_ref):
    h = lax.dot_general(wf_ref[...], f_ref[...], _DN0,
                        preferred_element_type=jnp.float32)
    h = h + lax.dot_general(wd_ref[...], d_ref[...], _DN0,
                            preferred_element_type=jnp.float32)
    h = h + lax.dot_general(wc_ref[...], c_ref[...], _DN0,
                            preferred_element_type=jnp.float32)
    e4 = e_ref[...]
    ne = we_ref.shape[0]
    h_e = jnp.concatenate(
        [lax.dot_general(we_ref[...],
                         lax.slice(e4, (0, a * ne), (e4.shape[0], (a + 1) * ne)),
                         _DN1, preferred_element_type=jnp.float32)
         for a in range(4)],
        axis=1)
    h = h + h_e
    h = h + b_ref[...]
    h = h * (1.0 / (1.0 + jnp.exp(-h)))
    mu = jnp.mean(h, axis=0, keepdims=True)
    hc = h - mu
    var = jnp.mean(hc * hc, axis=0, keepdims=True)
    out_ref[...] = hc * lax.rsqrt(var + 1e-5) * g_ref[...] + bet_ref[...]


def _sc_gather(oc, table, n_pad, n_chunks):
    return _make_sc_gather(n_pad, table.shape[1], n_chunks)(oc, table)


def _tc_body_carry(e_ref, f_ref, d_ref, c_ref, we_ref, wf_ref, wd_ref,
                   wc_ref, b_ref, g_ref, bet_ref, carry_ref, out_ref):
    del carry_ref
    _tc_body(e_ref, f_ref, d_ref, c_ref, we_ref, wf_ref, wd_ref, wc_ref,
             b_ref, g_ref, bet_ref, out_ref)


def kernel(op_code, features, configs, dim_features, table, W, b, gamma, beta):
    n = features.shape[0]
    ne = table.shape[1]
    nf = features.shape[1]
    nd = dim_features.shape[1]
    ncf = configs.shape[1]
    out_ch = W.shape[1]

    n_chunks = -(-n // (_NW * _CHUNK))
    n_pad = _NW * _CHUNK * n_chunks
    oc = op_code.reshape(-1).astype(jnp.int32)
    oc = jnp.concatenate([oc, jnp.zeros((n_pad - n,), jnp.int32)])

    s = jnp.arange(n_pad, dtype=jnp.int32)
    i = s // _BLK
    t = s % _BLK
    r = i * _BLK + (t % 4) * (_BLK // 4) + t // 4
    oc_perm = oc[r]

    w_e = W[:ne]
    w_f = W[ne:ne + nf]
    w_d = W[ne + nf:ne + nf + nd]
    w_c = W[ne + nf + nd:]

    f_t = features.T
    d_t = dim_features.T
    c_t = configs.T
    scalars = (b.reshape(-1, 1), gamma.reshape(-1, 1), beta.reshape(-1, 1))

    n_blocks = -(-n // _BLK)
    g_blocks = n_pad // _BLK
    bounds = [0, 6, 12, 18, g_blocks]

    out_t = None
    for si in range(len(bounds) - 1):
        b0, b1 = bounds[si], bounds[si + 1]
        n_sl = (b1 - b0) * _BLK
        e_s = _sc_gather(oc_perm[b0 * _BLK:b1 * _BLK], table, n_sl,
                         n_sl // (_NW * _CHUNK))
        e_s = e_s.reshape(n_sl * ne // 128, 128)

        tb1 = min(b1, n_blocks)
        col_block = lambda ch, b0=b0: pl.BlockSpec(
            (ch, _BLK), lambda i, b0=b0: (0, b0 + i))
        full = lambda a: pl.BlockSpec(a.shape, lambda i: (0, 0))
        in_specs = [
            pl.BlockSpec((_BLK * ne // 128, 128), lambda i: (i, 0)),
            col_block(nf),
            col_block(nd),
            col_block(ncf),
            full(w_e), full(w_f), full(w_d), full(w_c),
            pl.BlockSpec((out_ch, 1), lambda i: (0, 0)),
            pl.BlockSpec((out_ch, 1), lambda i: (0, 0)),
            pl.BlockSpec((out_ch, 1), lambda i: (0, 0)),
        ]
        args = (e_s, f_t, d_t, c_t, w_e, w_f, w_d, w_c) + scalars
        body = _tc_body
        alias = {}
        if out_t is not None:
            in_specs = in_specs + [pl.BlockSpec(memory_space=pl.ANY)]
            args = args + (out_t,)
            body = _tc_body_carry
            alias = {11: 0}
        out_t = pl.pallas_call(
            body,
            grid=(tb1 - b0,),
            in_specs=in_specs,
            out_specs=pl.BlockSpec((out_ch, _BLK),
                                   lambda i, b0=b0: (0, b0 + i)),
            out_shape=jax.ShapeDtypeStruct((out_ch, n), jnp.float32),
            input_output_aliases=alias,
        )(*args)
    return out_t.T
